# baseline (speedup 1.0000x reference)
import functools

import jax
import jax.numpy as jnp
from jax import lax
from jax.experimental import pallas as pl
from jax.experimental.pallas import tpu as pltpu
from jax.experimental.pallas import tpu_sc as plsc

NC = 2
NS = 16
NW = NC * NS
BATCH = 16384
D = 16
BPW = BATCH // NW
CH = 128
NCH = BPW // CH
NG = BPW // 16


def _rsqrt(x):
    i = plsc.bitcast(x, jnp.int32)
    y = plsc.bitcast(jnp.int32(0x5F3759DF) - (i >> 1), jnp.float32)
    for _ in range(3):
        y = y * (1.5 - 0.5 * x * y * y)
    return y


def _norm_inv(ss):
    r = _rsqrt(jnp.maximum(ss, jnp.float32(1e-24)))
    return jnp.where(ss > 1e-24, r, jnp.float32(1e12))


def _sqrt_nn(x):
    return x * _rsqrt(jnp.maximum(x, jnp.float32(1e-30)))


def kernel(users, adjacent_items, weak_items, strong_items,
           user_structure, item_structure):
    mesh = plsc.VectorSubcoreMesh(core_axis_name="c", subcore_axis_name="s")
    out_t = jax.ShapeDtypeStruct((BATCH,), jnp.float32)

    @functools.partial(
        pl.kernel,
        mesh=mesh,
        out_type=(out_t, out_t, out_t),
        compiler_params=pltpu.CompilerParams(
            needs_layout_passes=False, use_tc_tiling_on_sc=False),
        scratch_types=[
            pltpu.VMEM((BPW,), jnp.int32),
            pltpu.VMEM((BPW,), jnp.int32),
            pltpu.VMEM((BPW,), jnp.int32),
            pltpu.VMEM((BPW,), jnp.int32),
            pltpu.VMEM((BPW, D), jnp.float32),
            pltpu.VMEM((BPW, D), jnp.float32),
            pltpu.VMEM((BPW, D), jnp.float32),
            pltpu.VMEM((BPW, D), jnp.float32),
            pltpu.VMEM((BPW,), jnp.float32),
            pltpu.VMEM((BPW,), jnp.float32),
            pltpu.VMEM((BPW,), jnp.float32),
            pltpu.SemaphoreType.DMA,
        ],
    )
    def run(users_h, adj_h, weak_h, strong_h, utab_h, itab_h,
            oa_h, ow_h, os_h,
            iu, ia, iw, ist, ru, ra, rw, rs, oa, ow, osv, sem):
        wid = lax.axis_index("s") * NC + lax.axis_index("c")
        base = wid * BPW

        cps = [
            pltpu.async_copy(users_h.at[pl.ds(base, BPW)], iu, sem),
            pltpu.async_copy(adj_h.at[pl.ds(base, BPW)], ia, sem),
            pltpu.async_copy(weak_h.at[pl.ds(base, BPW)], iw, sem),
            pltpu.async_copy(strong_h.at[pl.ds(base, BPW)], ist, sem),
        ]
        for c in cps:
            c.wait()

        cps = []
        for j in range(NCH):
            sl = pl.ds(j * CH, CH)
            cps.append(pltpu.async_copy(utab_h.at[iu.at[sl]], ru.at[sl], sem))
            cps.append(pltpu.async_copy(itab_h.at[ia.at[sl]], ra.at[sl], sem))
            cps.append(pltpu.async_copy(itab_h.at[iw.at[sl]], rw.at[sl], sem))
            cps.append(pltpu.async_copy(itab_h.at[ist.at[sl]], rs.at[sl], sem))
        for c in cps:
            c.wait()

        iota = lax.iota(jnp.int32, 16)
        eps = jnp.float32(1e-6)
        deps2 = jnp.float32(D * 1e-6 * 1e-6)
        zero = jnp.zeros((16,), jnp.float32)

        def group(g, carry):
            row0 = g * 16
            rows = row0 + iota
            ssu = ssa = ssw = sss = zero
            su = sa = sw = ss_ = zero
            dua = duw = dus = zero
            for d in range(D):
                cd = jnp.full((16,), d, jnp.int32)
                u = plsc.load_gather(ru, [rows, cd])
                a = plsc.load_gather(ra, [rows, cd])
                w = plsc.load_gather(rw, [rows, cd])
                s = plsc.load_gather(rs, [rows, cd])
                ssu = ssu + u * u
                ssa = ssa + a * a
                ssw = ssw + w * w
                sss = sss + s * s
                su = su + u
                sa = sa + a
                sw = sw + w
                ss_ = ss_ + s
                dua = dua + u * a
                duw = duw + u * w
                dus = dus + u * s
            nu = _norm_inv(ssu)
            na = _norm_inv(ssa)
            nw_ = _norm_inv(ssw)
            ns = _norm_inv(sss)
            squ = ssu * nu * nu
            suu = su * nu
            for (ss2, n2, s2, dot, out_ref) in (
                    (ssa, na, sa, dua, oa),
                    (ssw, nw_, sw, duw, ow),
                    (sss, ns, ss_, dus, osv)):
                d2 = (squ + ss2 * n2 * n2 + deps2
                      - 2.0 * dot * nu * n2
                      + (2.0 * eps) * (suu - s2 * n2))
                dist = _sqrt_nn(jnp.maximum(d2, jnp.float32(0.0)))
                out_ref[pl.ds(row0, 16)] = 1.0 - 0.5 * dist
            return carry

        lax.fori_loop(0, NG, group, 0)

        pltpu.sync_copy(oa, oa_h.at[pl.ds(base, BPW)])
        pltpu.sync_copy(ow, ow_h.at[pl.ds(base, BPW)])
        pltpu.sync_copy(osv, os_h.at[pl.ds(base, BPW)])

    return run(users, adjacent_items, weak_items, strong_items,
               user_structure, item_structure)
